# Initial kernel scaffold; baseline (speedup 1.0000x reference)
import jax
import jax.numpy as jnp
from jax.experimental import pallas as pl


def kernel(x, edge_index, edge_attr, p, mean_vec, std_vec, mean_edge_vec, std_edge_vec, ne_W1, ne_b1, ne_W2, ne_b2, ne_g, ne_b, ee_W1, ee_b1, ee_W2, ee_b2, ee_g, ee_b, pe_W1, pe_b1, pe_W2, pe_b2, pe_g, pe_b, pn_W1, pn_b1, pn_W2, pn_b2, pn_g, pn_b, dec_W1, dec_b1, dec_W2, dec_b2):
    raise NotImplementedError("write your pallas kernel here")


# speedup vs baseline: 1.2317x; 1.2317x over previous
import functools

import jax
import jax.numpy as jnp
from jax import lax
from jax.experimental import pallas as pl
from jax.experimental.pallas import tpu as pltpu

_N = 10000
_E = 320000
_H = 128
_NB = 1000
_EB = 4000


def _ln(t, g, b):
    mu = jnp.mean(t, axis=-1, keepdims=True)
    var = jnp.mean((t - mu) ** 2, axis=-1, keepdims=True)
    return (t - mu) * lax.rsqrt(var + 1e-5) * g + b


def _row_spec(shape):
    nd = len(shape)
    return pl.BlockSpec(shape, lambda i: (i,) + (0,) * (nd - 1))


def _full_spec(shape):
    nd = len(shape)
    return pl.BlockSpec(shape, lambda i: (0,) * nd)


def _enc_body(x_ref, w1, b1, w2, b2, g, b, wpd, bpd, wps,
              h_ref, pd_ref, ps_ref):
    t = jnp.maximum(x_ref[...] @ w1[...] + b1[...], 0.0)
    t = t @ w2[...] + b2[...]
    h = _ln(t, g[...], b[...])
    h_ref[...] = h
    pd_ref[...] = h @ wpd[...] + bpd[...]
    ps_ref[...] = h @ wps[...]


def _edge1_body(g_ref, ea_ref, ew1, eb1, ew2, eb2, eg, ebb,
                pw1e, pw2, pb2, pg, pb, out_ref):
    he = _ln(jnp.maximum(ea_ref[...] @ ew1[...] + eb1[...], 0.0)
             @ ew2[...] + eb2[...], eg[...], ebb[...])
    t = jnp.maximum(g_ref[...] + he @ pw1e[...], 0.0)
    out_ref[...] = _ln(t @ pw2[...] + pb2[...], pg[...], pb[...]) + he


def _edge2_body(g_ref, he_ref, pw1e, pw2, pb2, pg, pb, out_ref):
    he = he_ref[...]
    t = jnp.maximum(g_ref[...] + he @ pw1e[...], 0.0)
    out_ref[...] = _ln(t @ pw2[...] + pb2[...], pg[...], pb[...]) + he


def _node_body(h_ref, a_ref, w1h, w1a, b1, w2, b2, g, b, wpd, bpd, wps,
               h1_ref, pd_ref, ps_ref):
    h = h_ref[...]
    t = jnp.maximum(h @ w1h[...] + a_ref[...] @ w1a[...] + b1[...], 0.0)
    t = t @ w2[...] + b2[...]
    h1 = h + _ln(t, g[...], b[...])
    h1_ref[...] = h1
    pd_ref[...] = h1 @ wpd[...] + bpd[...]
    ps_ref[...] = h1 @ wps[...]


def _final_body(h_ref, a_ref, w1h, w1a, b1, w2, b2, g, b,
                dw1, db1, dw2, db2, out_ref):
    h = h_ref[...]
    t = jnp.maximum(h @ w1h[...] + a_ref[...] @ w1a[...] + b1[...], 0.0)
    t = t @ w2[...] + b2[...]
    h2 = h + _ln(t, g[...], b[...])
    o = jnp.maximum(h2 @ dw1[...] + db1[...], 0.0)
    out_ref[...] = o @ dw2[...] + db2[...]


def _nodes_call(body, n_out, ins):
    shapes = [jax.ShapeDtypeStruct((_N, _H), jnp.float32)] * n_out
    specs = [_row_spec((_NB, _H))] * n_out
    return pl.pallas_call(
        body,
        grid=(_N // _NB,),
        in_specs=[_row_spec((_NB,) + x.shape[1:]) if x.shape[0] == _N
                  else _full_spec(x.shape) for x in ins],
        out_specs=specs if n_out > 1 else specs[0],
        out_shape=shapes if n_out > 1 else shapes[0],
    )(*ins)


def _edges_call(body, ins):
    in_specs = []
    for x in ins:
        if x.shape[0] == _E:
            in_specs.append(_row_spec((_EB,) + x.shape[1:]))
        else:
            in_specs.append(_full_spec(x.shape))
    return pl.pallas_call(
        body,
        grid=(_E // _EB,),
        in_specs=in_specs,
        out_specs=_row_spec((_EB, _H)),
        out_shape=jax.ShapeDtypeStruct((_E, _H), jnp.float32),
    )(*ins)


def kernel(x, edge_index, edge_attr, p, mean_vec, std_vec, mean_edge_vec,
           std_edge_vec, ne_W1, ne_b1, ne_W2, ne_b2, ne_g, ne_b,
           ee_W1, ee_b1, ee_W2, ee_b2, ee_g, ee_b,
           pe_W1, pe_b1, pe_W2, pe_b2, pe_g, pe_b,
           pn_W1, pn_b1, pn_W2, pn_b2, pn_g, pn_b,
           dec_W1, dec_b1, dec_W2, dec_b2):
    H = _H
    f32 = jnp.float32
    src = edge_index[0]
    dst = edge_index[1]

    nW1 = ne_W1 / std_vec[:, None]
    nb1 = (ne_b1 - (mean_vec / std_vec) @ ne_W1).reshape(1, H)
    eW1 = ee_W1 / std_edge_vec[:, None]
    eb1 = (ee_b1 - (mean_edge_vec / std_edge_vec) @ ee_W1).reshape(1, H)

    wpd, wps, wpe = pe_W1[:H], pe_W1[H:2 * H], pe_W1[2 * H:]
    w1h, w1a = pn_W1[:H], pn_W1[H:]

    r2 = lambda v: v.reshape(1, -1).astype(f32)
    dW2p = jnp.zeros((H, H), f32).at[:, :dec_W2.shape[1]].set(dec_W2)
    db2p = jnp.zeros((1, H), f32).at[0, :dec_b2.shape[0]].set(dec_b2)

    h, Pd, Ps = _nodes_call(
        _enc_body, 3,
        (x, nW1, nb1, ne_W2, r2(ne_b2), r2(ne_g), r2(ne_b),
         wpd, r2(pe_b1), wps))

    def gather(Pd, Ps):
        return jnp.take(Pd, dst, axis=0) + jnp.take(Ps, src, axis=0)

    def scatter(he):
        return jax.ops.segment_sum(he, src, num_segments=_N)

    G = gather(Pd, Ps)
    he1 = _edges_call(
        _edge1_body,
        (G, edge_attr, eW1, eb1, ee_W2, r2(ee_b2), r2(ee_g), r2(ee_b),
         wpe, pe_W2, r2(pe_b2), r2(pe_g), r2(pe_b)))
    agg1 = scatter(he1)
    h1, Pd1, Ps1 = _nodes_call(
        _node_body, 3,
        (h, agg1, w1h, w1a, r2(pn_b1), pn_W2, r2(pn_b2), r2(pn_g), r2(pn_b),
         wpd, r2(pe_b1), wps))

    G1 = gather(Pd1, Ps1)
    he2 = _edges_call(
        _edge2_body,
        (G1, he1, wpe, pe_W2, r2(pe_b2), r2(pe_g), r2(pe_b)))
    agg2 = scatter(he2)

    out = _nodes_call(
        _final_body, 1,
        (h1, agg2, w1h, w1a, r2(pn_b1), pn_W2, r2(pn_b2), r2(pn_g), r2(pn_b),
         dec_W1, r2(dec_b1), dW2p, db2p))
    return out[:, :dec_W2.shape[1]]


# speedup vs baseline: 2.3060x; 1.8722x over previous
import functools

import jax
import jax.numpy as jnp
from jax import lax
from jax.experimental import pallas as pl
from jax.experimental.pallas import tpu as pltpu
from jax.experimental.pallas import tpu_sc as plsc

_N = 10000
_E = 320000
_H = 128
_NB = 1000
_EB = 4000


def _ln(t, g, b):
    mu = jnp.mean(t, axis=-1, keepdims=True)
    var = jnp.mean((t - mu) ** 2, axis=-1, keepdims=True)
    return (t - mu) * lax.rsqrt(var + 1e-5) * g + b


def _row_spec(shape):
    nd = len(shape)
    return pl.BlockSpec(shape, lambda i: (i,) + (0,) * (nd - 1))


def _full_spec(shape):
    nd = len(shape)
    return pl.BlockSpec(shape, lambda i: (0,) * nd)


def _enc_body(x_ref, w1, b1, w2, b2, g, b, wpd, bpd, wps,
              h_ref, pd_ref, ps_ref):
    t = jnp.maximum(x_ref[...] @ w1[...] + b1[...], 0.0)
    t = t @ w2[...] + b2[...]
    h = _ln(t, g[...], b[...])
    h_ref[...] = h
    pd_ref[...] = h @ wpd[...] + bpd[...]
    ps_ref[...] = h @ wps[...]


def _edge1_body(g_ref, ea_ref, ew1, eb1, ew2, eb2, eg, ebb,
                pw1e, pw2, pb2, pg, pb, out_ref):
    he = _ln(jnp.maximum(ea_ref[...] @ ew1[...] + eb1[...], 0.0)
             @ ew2[...] + eb2[...], eg[...], ebb[...])
    t = jnp.maximum(g_ref[...] + he @ pw1e[...], 0.0)
    out_ref[...] = _ln(t @ pw2[...] + pb2[...], pg[...], pb[...]) + he


def _edge2_body(g_ref, he_ref, pw1e, pw2, pb2, pg, pb, out_ref):
    he = he_ref[...]
    t = jnp.maximum(g_ref[...] + he @ pw1e[...], 0.0)
    out_ref[...] = _ln(t @ pw2[...] + pb2[...], pg[...], pb[...]) + he


def _node_body(h_ref, a_ref, w1h, w1a, b1, w2, b2, g, b, wpd, bpd, wps,
               h1_ref, pd_ref, ps_ref):
    h = h_ref[...]
    t = jnp.maximum(h @ w1h[...] + a_ref[...] @ w1a[...] + b1[...], 0.0)
    t = t @ w2[...] + b2[...]
    h1 = h + _ln(t, g[...], b[...])
    h1_ref[...] = h1
    pd_ref[...] = h1 @ wpd[...] + bpd[...]
    ps_ref[...] = h1 @ wps[...]


def _final_body(h_ref, a_ref, w1h, w1a, b1, w2, b2, g, b,
                dw1, db1, dw2, db2, out_ref):
    h = h_ref[...]
    t = jnp.maximum(h @ w1h[...] + a_ref[...] @ w1a[...] + b1[...], 0.0)
    t = t @ w2[...] + b2[...]
    h2 = h + _ln(t, g[...], b[...])
    o = jnp.maximum(h2 @ dw1[...] + db1[...], 0.0)
    out_ref[...] = o @ dw2[...] + db2[...]


_NW = 32
_CH = 640
_NCH = _E // _CH
_HCH = _CH // 2


def _sc_gather(Pd, Ps, dst2d, src2d):
    mesh = plsc.VectorSubcoreMesh(core_axis_name="c", subcore_axis_name="s")

    @functools.partial(
        pl.kernel, mesh=mesh,
        out_type=jax.ShapeDtypeStruct((_E, _H), jnp.float32),
        scratch_types=[
            pltpu.VMEM((8, 80), jnp.int32),
            pltpu.VMEM((8, 80), jnp.int32),
            pltpu.VMEM((_HCH, _H), jnp.float32),
            pltpu.VMEM((_HCH, _H), jnp.float32),
            pltpu.SemaphoreType.DMA,
        ],
    )
    def k(pd_hbm, ps_hbm, dst_hbm, src_hbm, out_hbm, idd, ids, rd, rs, sem):
        wid = lax.axis_index("s") * 2 + lax.axis_index("c")
        nfull = _NCH // _NW
        nch_w = jnp.where(wid < _NCH - nfull * _NW, nfull + 1, nfull)

        def body(g, _):
            c = wid + _NW * g
            pltpu.sync_copy(dst_hbm.at[pl.ds(c * 8, 8)], idd)
            pltpu.sync_copy(src_hbm.at[pl.ds(c * 8, 8)], ids)
            for half in range(2):
                cps = []
                for j in range(4):
                    cps.append(pltpu.async_copy(
                        pd_hbm.at[idd.at[half * 4 + j]],
                        rd.at[pl.ds(j * 80, 80)], sem))
                    cps.append(pltpu.async_copy(
                        ps_hbm.at[ids.at[half * 4 + j]],
                        rs.at[pl.ds(j * 80, 80)], sem))
                for cp in cps:
                    cp.wait()

                def addrow(r, carry):
                    for kk in range(_H // 16):
                        sl = pl.ds(kk * 16, 16)
                        rd[r, sl] = rd[r, sl] + rs[r, sl]
                    return carry

                lax.fori_loop(0, _HCH, addrow, 0)
                pltpu.sync_copy(
                    rd, out_hbm.at[pl.ds(c * _CH + half * _HCH, _HCH)])
            return 0

        lax.fori_loop(0, nch_w, body, 0)

    return k(Pd, Ps, dst2d, src2d)


def _nodes_call(body, n_out, ins):
    shapes = [jax.ShapeDtypeStruct((_N, _H), jnp.float32)] * n_out
    specs = [_row_spec((_NB, _H))] * n_out
    return pl.pallas_call(
        body,
        grid=(_N // _NB,),
        in_specs=[_row_spec((_NB,) + x.shape[1:]) if x.shape[0] == _N
                  else _full_spec(x.shape) for x in ins],
        out_specs=specs if n_out > 1 else specs[0],
        out_shape=shapes if n_out > 1 else shapes[0],
    )(*ins)


def _edges_call(body, ins):
    in_specs = []
    for x in ins:
        if x.shape[0] == _E:
            in_specs.append(_row_spec((_EB,) + x.shape[1:]))
        else:
            in_specs.append(_full_spec(x.shape))
    return pl.pallas_call(
        body,
        grid=(_E // _EB,),
        in_specs=in_specs,
        out_specs=_row_spec((_EB, _H)),
        out_shape=jax.ShapeDtypeStruct((_E, _H), jnp.float32),
    )(*ins)


def kernel(x, edge_index, edge_attr, p, mean_vec, std_vec, mean_edge_vec,
           std_edge_vec, ne_W1, ne_b1, ne_W2, ne_b2, ne_g, ne_b,
           ee_W1, ee_b1, ee_W2, ee_b2, ee_g, ee_b,
           pe_W1, pe_b1, pe_W2, pe_b2, pe_g, pe_b,
           pn_W1, pn_b1, pn_W2, pn_b2, pn_g, pn_b,
           dec_W1, dec_b1, dec_W2, dec_b2):
    H = _H
    f32 = jnp.float32
    src = edge_index[0]
    dst = edge_index[1]

    nW1 = ne_W1 / std_vec[:, None]
    nb1 = (ne_b1 - (mean_vec / std_vec) @ ne_W1).reshape(1, H)
    eW1 = ee_W1 / std_edge_vec[:, None]
    eb1 = (ee_b1 - (mean_edge_vec / std_edge_vec) @ ee_W1).reshape(1, H)

    wpd, wps, wpe = pe_W1[:H], pe_W1[H:2 * H], pe_W1[2 * H:]
    w1h, w1a = pn_W1[:H], pn_W1[H:]

    r2 = lambda v: v.reshape(1, -1).astype(f32)
    dW2p = jnp.zeros((H, H), f32).at[:, :dec_W2.shape[1]].set(dec_W2)
    db2p = jnp.zeros((1, H), f32).at[0, :dec_b2.shape[0]].set(dec_b2)

    h, Pd, Ps = _nodes_call(
        _enc_body, 3,
        (x, nW1, nb1, ne_W2, r2(ne_b2), r2(ne_g), r2(ne_b),
         wpd, r2(pe_b1), wps))

    dst2d = dst.reshape(_E // 80, 80)
    src2d = src.reshape(_E // 80, 80)

    def gather(Pd, Ps):
        return _sc_gather(Pd, Ps, dst2d, src2d)

    def scatter(he):
        return jax.ops.segment_sum(he, src, num_segments=_N)

    G = gather(Pd, Ps)
    he1 = _edges_call(
        _edge1_body,
        (G, edge_attr, eW1, eb1, ee_W2, r2(ee_b2), r2(ee_g), r2(ee_b),
         wpe, pe_W2, r2(pe_b2), r2(pe_g), r2(pe_b)))
    agg1 = scatter(he1)
    h1, Pd1, Ps1 = _nodes_call(
        _node_body, 3,
        (h, agg1, w1h, w1a, r2(pn_b1), pn_W2, r2(pn_b2), r2(pn_g), r2(pn_b),
         wpd, r2(pe_b1), wps))

    G1 = gather(Pd1, Ps1)
    he2 = _edges_call(
        _edge2_body,
        (G1, he1, wpe, pe_W2, r2(pe_b2), r2(pe_g), r2(pe_b)))
    agg2 = scatter(he2)

    out = _nodes_call(
        _final_body, 1,
        (h1, agg2, w1h, w1a, r2(pn_b1), pn_W2, r2(pn_b2), r2(pn_g), r2(pn_b),
         dec_W1, r2(dec_b1), dW2p, db2p))
    return out[:, :dec_W2.shape[1]]


# speedup vs baseline: 4.2725x; 1.8528x over previous
import functools

import jax
import jax.numpy as jnp
from jax import lax
from jax.experimental import pallas as pl
from jax.experimental.pallas import tpu as pltpu
from jax.experimental.pallas import tpu_sc as plsc

_N = 10000
_E = 320000
_H = 128
_NB = 1000
_EB = 4000


def _ln(t, g, b):
    mu = jnp.mean(t, axis=-1, keepdims=True)
    var = jnp.mean((t - mu) ** 2, axis=-1, keepdims=True)
    return (t - mu) * lax.rsqrt(var + 1e-5) * g + b


def _row_spec(shape):
    nd = len(shape)
    return pl.BlockSpec(shape, lambda i: (i,) + (0,) * (nd - 1))


def _full_spec(shape):
    nd = len(shape)
    return pl.BlockSpec(shape, lambda i: (0,) * nd)


def _enc_body(x_ref, w1, b1, w2, b2, g, b, wpd, bpd, wps,
              h_ref, pd_ref, ps_ref):
    t = jnp.maximum(x_ref[...] @ w1[...] + b1[...], 0.0)
    t = t @ w2[...] + b2[...]
    h = _ln(t, g[...], b[...])
    h_ref[...] = h
    pd_ref[...] = h @ wpd[...] + bpd[...]
    ps_ref[...] = h @ wps[...]


def _edge1_body(g_ref, ea_ref, ew1, eb1, ew2, eb2, eg, ebb,
                pw1e, pw2, pb2, pg, pb, out_ref):
    he = _ln(jnp.maximum(ea_ref[...] @ ew1[...] + eb1[...], 0.0)
             @ ew2[...] + eb2[...], eg[...], ebb[...])
    t = jnp.maximum(g_ref[...] + he @ pw1e[...], 0.0)
    out_ref[...] = _ln(t @ pw2[...] + pb2[...], pg[...], pb[...]) + he


def _edge2_body(g_ref, he_ref, pw1e, pw2, pb2, pg, pb, out_ref):
    he = he_ref[...]
    t = jnp.maximum(g_ref[...] + he @ pw1e[...], 0.0)
    out_ref[...] = _ln(t @ pw2[...] + pb2[...], pg[...], pb[...]) + he


def _node_body(h_ref, a_ref, w1h, w1a, b1, w2, b2, g, b, wpd, bpd, wps,
               h1_ref, pd_ref, ps_ref):
    h = h_ref[...]
    agg = a_ref[0] + a_ref[1]
    t = jnp.maximum(h @ w1h[...] + agg @ w1a[...] + b1[...], 0.0)
    t = t @ w2[...] + b2[...]
    h1 = h + _ln(t, g[...], b[...])
    h1_ref[...] = h1
    pd_ref[...] = h1 @ wpd[...] + bpd[...]
    ps_ref[...] = h1 @ wps[...]


def _final_body(h_ref, a_ref, w1h, w1a, b1, w2, b2, g, b,
                dw1, db1, dw2, db2, out_ref):
    h = h_ref[...]
    agg = a_ref[0] + a_ref[1]
    t = jnp.maximum(h @ w1h[...] + agg @ w1a[...] + b1[...], 0.0)
    t = t @ w2[...] + b2[...]
    h2 = h + _ln(t, g[...], b[...])
    o = jnp.maximum(h2 @ dw1[...] + db1[...], 0.0)
    out_ref[...] = o @ dw2[...] + db2[...]


_NW = 32
_CH = 640
_NCH = _E // _CH
_HCH = _CH // 2


def _sc_gather(Pd, Ps, dst2d, src2d):
    mesh = plsc.VectorSubcoreMesh(core_axis_name="c", subcore_axis_name="s")

    @functools.partial(
        pl.kernel, mesh=mesh,
        out_type=jax.ShapeDtypeStruct((_E, _H), jnp.float32),
        scratch_types=[
            pltpu.VMEM((8, 80), jnp.int32),
            pltpu.VMEM((8, 80), jnp.int32),
            pltpu.VMEM((_HCH, _H), jnp.float32),
            pltpu.VMEM((_HCH, _H), jnp.float32),
            pltpu.SemaphoreType.DMA,
        ],
    )
    def k(pd_hbm, ps_hbm, dst_hbm, src_hbm, out_hbm, idd, ids, rd, rs, sem):
        wid = lax.axis_index("s") * 2 + lax.axis_index("c")
        nfull = _NCH // _NW
        nch_w = jnp.where(wid < _NCH - nfull * _NW, nfull + 1, nfull)

        def body(g, _):
            c = wid + _NW * g
            pltpu.sync_copy(dst_hbm.at[pl.ds(c * 8, 8)], idd)
            pltpu.sync_copy(src_hbm.at[pl.ds(c * 8, 8)], ids)
            for half in range(2):
                cps = []
                for j in range(4):
                    cps.append(pltpu.async_copy(
                        pd_hbm.at[idd.at[half * 4 + j]],
                        rd.at[pl.ds(j * 80, 80)], sem))
                    cps.append(pltpu.async_copy(
                        ps_hbm.at[ids.at[half * 4 + j]],
                        rs.at[pl.ds(j * 80, 80)], sem))
                for cp in cps:
                    cp.wait()

                def addrow(r, carry):
                    for kk in range(_H // 16):
                        sl = pl.ds(kk * 16, 16)
                        rd[r, sl] = rd[r, sl] + rs[r, sl]
                    return carry

                lax.fori_loop(0, _HCH, addrow, 0)
                pltpu.sync_copy(
                    rd, out_hbm.at[pl.ds(c * _CH + half * _HCH, _HCH)])
            return 0

        lax.fori_loop(0, nch_w, body, 0)

    return k(Pd, Ps, dst2d, src2d)


def _sc_scatter(he, src2d, zrows):
    mesh = plsc.VectorSubcoreMesh(core_axis_name="c", subcore_axis_name="s")

    @functools.partial(
        pl.kernel, mesh=mesh,
        out_type=jax.ShapeDtypeStruct((2, _N, _H), jnp.float32),
        scratch_types=[
            pltpu.VMEM((8, 80), jnp.int32),
            pltpu.VMEM((_HCH, _H), jnp.float32),
            pltpu.VMEM_SHARED((_N, _H), jnp.float32),
        ],
    )
    def k(he_hbm, src_hbm, z_hbm, out_hbm, ids, rows, acc):
        cid = lax.axis_index("c")
        sid = lax.axis_index("s")
        wid = sid * 2 + cid
        rbase = sid * 624
        pltpu.sync_copy(z_hbm.at[pl.ds(rbase, 624)],
                        acc.at[pl.ds(rbase, 624)])

        @pl.when(sid == 0)
        def _():
            pltpu.sync_copy(z_hbm.at[pl.ds(16 * 624, _N - 16 * 624)],
                            acc.at[pl.ds(16 * 624, _N - 16 * 624)])

        plsc.subcore_barrier()

        nfull = _NCH // _NW
        nch_w = jnp.where(wid < _NCH - nfull * _NW, nfull + 1, nfull)

        def body(g, _):
            c = wid + _NW * g
            pltpu.sync_copy(src_hbm.at[pl.ds(c * 8, 8)], ids)
            for half in range(2):
                pltpu.sync_copy(
                    he_hbm.at[pl.ds(c * _CH + half * _HCH, _HCH)], rows)
                for j in range(4):
                    pltpu.sync_copy(rows.at[pl.ds(j * 80, 80)],
                                    acc.at[ids.at[half * 4 + j]], add=True)
            return 0

        lax.fori_loop(0, nch_w, body, 0)
        plsc.subcore_barrier()
        pltpu.sync_copy(acc.at[pl.ds(rbase, 624)],
                        out_hbm.at[cid, pl.ds(rbase, 624)])

        @pl.when(sid == 0)
        def _():
            pltpu.sync_copy(acc.at[pl.ds(16 * 624, _N - 16 * 624)],
                            out_hbm.at[cid, pl.ds(16 * 624, _N - 16 * 624)])

    return k(he, src2d, zrows)


def _nodes_call(body, n_out, ins):
    shapes = [jax.ShapeDtypeStruct((_N, _H), jnp.float32)] * n_out
    specs = [_row_spec((_NB, _H))] * n_out
    return pl.pallas_call(
        body,
        grid=(_N // _NB,),
        in_specs=[pl.BlockSpec((2, _NB, _H), lambda i: (0, i, 0))
                  if x.ndim == 3
                  else (_row_spec((_NB,) + x.shape[1:]) if x.shape[0] == _N
                        else _full_spec(x.shape)) for x in ins],
        out_specs=specs if n_out > 1 else specs[0],
        out_shape=shapes if n_out > 1 else shapes[0],
    )(*ins)


def _edges_call(body, ins):
    in_specs = []
    for x in ins:
        if x.shape[0] == _E:
            in_specs.append(_row_spec((_EB,) + x.shape[1:]))
        else:
            in_specs.append(_full_spec(x.shape))
    return pl.pallas_call(
        body,
        grid=(_E // _EB,),
        in_specs=in_specs,
        out_specs=_row_spec((_EB, _H)),
        out_shape=jax.ShapeDtypeStruct((_E, _H), jnp.float32),
    )(*ins)


def kernel(x, edge_index, edge_attr, p, mean_vec, std_vec, mean_edge_vec,
           std_edge_vec, ne_W1, ne_b1, ne_W2, ne_b2, ne_g, ne_b,
           ee_W1, ee_b1, ee_W2, ee_b2, ee_g, ee_b,
           pe_W1, pe_b1, pe_W2, pe_b2, pe_g, pe_b,
           pn_W1, pn_b1, pn_W2, pn_b2, pn_g, pn_b,
           dec_W1, dec_b1, dec_W2, dec_b2):
    H = _H
    f32 = jnp.float32
    src = edge_index[0]
    dst = edge_index[1]

    nW1 = ne_W1 / std_vec[:, None]
    nb1 = (ne_b1 - (mean_vec / std_vec) @ ne_W1).reshape(1, H)
    eW1 = ee_W1 / std_edge_vec[:, None]
    eb1 = (ee_b1 - (mean_edge_vec / std_edge_vec) @ ee_W1).reshape(1, H)

    wpd, wps, wpe = pe_W1[:H], pe_W1[H:2 * H], pe_W1[2 * H:]
    w1h, w1a = pn_W1[:H], pn_W1[H:]

    r2 = lambda v: v.reshape(1, -1).astype(f32)
    dW2p = jnp.zeros((H, H), f32).at[:, :dec_W2.shape[1]].set(dec_W2)
    db2p = jnp.zeros((1, H), f32).at[0, :dec_b2.shape[0]].set(dec_b2)

    h, Pd, Ps = _nodes_call(
        _enc_body, 3,
        (x, nW1, nb1, ne_W2, r2(ne_b2), r2(ne_g), r2(ne_b),
         wpd, r2(pe_b1), wps))

    dst2d = dst.reshape(_E // 80, 80)
    src2d = src.reshape(_E // 80, 80)

    def gather(Pd, Ps):
        return _sc_gather(Pd, Ps, dst2d, src2d)

    zrows = jnp.zeros((_N, _H), f32)

    def scatter(he):
        return _sc_scatter(he, src2d, zrows)

    G = gather(Pd, Ps)
    he1 = _edges_call(
        _edge1_body,
        (G, edge_attr, eW1, eb1, ee_W2, r2(ee_b2), r2(ee_g), r2(ee_b),
         wpe, pe_W2, r2(pe_b2), r2(pe_g), r2(pe_b)))
    agg1 = scatter(he1)
    h1, Pd1, Ps1 = _nodes_call(
        _node_body, 3,
        (h, agg1, w1h, w1a, r2(pn_b1), pn_W2, r2(pn_b2), r2(pn_g), r2(pn_b),
         wpd, r2(pe_b1), wps))

    G1 = gather(Pd1, Ps1)
    he2 = _edges_call(
        _edge2_body,
        (G1, he1, wpe, pe_W2, r2(pe_b2), r2(pe_g), r2(pe_b)))
    agg2 = scatter(he2)

    out = _nodes_call(
        _final_body, 1,
        (h1, agg2, w1h, w1a, r2(pn_b1), pn_W2, r2(pn_b2), r2(pn_g), r2(pn_b),
         dec_W1, r2(dec_b1), dW2p, db2p))
    return out[:, :dec_W2.shape[1]]


# speedup vs baseline: 4.9869x; 1.1672x over previous
import functools

import jax
import jax.numpy as jnp
from jax import lax
from jax.experimental import pallas as pl
from jax.experimental.pallas import tpu as pltpu
from jax.experimental.pallas import tpu_sc as plsc

_N = 10000
_E = 320000
_H = 128
_NB = 1000
_EB = 4000


def _ln(t, g, b):
    mu = jnp.mean(t, axis=-1, keepdims=True)
    var = jnp.mean((t - mu) ** 2, axis=-1, keepdims=True)
    return (t - mu) * lax.rsqrt(var + 1e-5) * g + b


def _row_spec(shape):
    nd = len(shape)
    return pl.BlockSpec(shape, lambda i: (i,) + (0,) * (nd - 1))


def _full_spec(shape):
    nd = len(shape)
    return pl.BlockSpec(shape, lambda i: (0,) * nd)


def _enc_body(x_ref, w1, b1, w2, b2, g, b, wpd, bpd, wps,
              h_ref, pd_ref, ps_ref):
    t = jnp.maximum(x_ref[...] @ w1[...] + b1[...], 0.0)
    t = t @ w2[...] + b2[...]
    h = _ln(t, g[...], b[...])
    h_ref[...] = h
    pd_ref[...] = h @ wpd[...] + bpd[...]
    ps_ref[...] = h @ wps[...]


def _edge1_body(g_ref, ea_ref, ew1, eb1, ew2, eb2, eg, ebb,
                pw1e, pw2, pb2, pg, pb, out_ref):
    he = _ln(jnp.maximum(ea_ref[...] @ ew1[...] + eb1[...], 0.0)
             @ ew2[...] + eb2[...], eg[...], ebb[...])
    t = jnp.maximum(g_ref[...] + he @ pw1e[...], 0.0)
    out_ref[...] = _ln(t @ pw2[...] + pb2[...], pg[...], pb[...]) + he


def _edge2_body(g_ref, he_ref, pw1e, pw2, pb2, pg, pb, out_ref):
    he = he_ref[...]
    t = jnp.maximum(g_ref[...] + he @ pw1e[...], 0.0)
    out_ref[...] = _ln(t @ pw2[...] + pb2[...], pg[...], pb[...]) + he


def _node_body(h_ref, a_ref, w1h, w1a, b1, w2, b2, g, b, wpd, bpd, wps,
               h1_ref, pd_ref, ps_ref):
    h = h_ref[...]
    agg = a_ref[0] + a_ref[1]
    t = jnp.maximum(h @ w1h[...] + agg @ w1a[...] + b1[...], 0.0)
    t = t @ w2[...] + b2[...]
    h1 = h + _ln(t, g[...], b[...])
    h1_ref[...] = h1
    pd_ref[...] = h1 @ wpd[...] + bpd[...]
    ps_ref[...] = h1 @ wps[...]


def _final_body(h_ref, a_ref, w1h, w1a, b1, w2, b2, g, b,
                dw1, db1, dw2, db2, out_ref):
    h = h_ref[...]
    agg = a_ref[0] + a_ref[1]
    t = jnp.maximum(h @ w1h[...] + agg @ w1a[...] + b1[...], 0.0)
    t = t @ w2[...] + b2[...]
    h2 = h + _ln(t, g[...], b[...])
    o = jnp.maximum(h2 @ dw1[...] + db1[...], 0.0)
    out_ref[...] = o @ dw2[...] + db2[...]


_NW = 32
_CH = 640
_NCH = _E // _CH
_HCH = _CH // 2


_QR = 160


def _sc_gather(Pd, Ps, dst2d, src2d):
    mesh = plsc.VectorSubcoreMesh(core_axis_name="c", subcore_axis_name="s")
    nfull = _NCH // _NW
    nrem = _NCH - nfull * _NW

    @functools.partial(
        pl.kernel, mesh=mesh,
        out_type=jax.ShapeDtypeStruct((_E, _H), jnp.float32),
        scratch_types=[
            pltpu.VMEM((128, 80), jnp.int32),
            pltpu.VMEM((128, 80), jnp.int32),
            pltpu.VMEM((_QR, _H), jnp.float32),
            pltpu.VMEM((_QR, _H), jnp.float32),
            pltpu.VMEM((_QR, _H), jnp.float32),
            pltpu.VMEM((_QR, _H), jnp.float32),
            pltpu.SemaphoreType.DMA,
            pltpu.SemaphoreType.DMA,
        ],
    )
    def k(pd_hbm, ps_hbm, dst_hbm, src_hbm, out_hbm,
          idd, ids, rd0, rs0, rd1, rs1, sem0, sem1):
        wid = lax.axis_index("s") * 2 + lax.axis_index("c")
        nch_w = jnp.where(wid < nrem, nfull + 1, nfull)
        nq = nch_w * 4
        bufs = ((rd0, rs0, sem0), (rd1, rs1, sem1))

        cps = []
        for g in range(nfull + 1):
            c = wid + _NW * g
            cps.append(pltpu.async_copy(
                dst_hbm.at[pl.ds(c * 8, 8)], idd.at[pl.ds(g * 8, 8)], sem0))
            cps.append(pltpu.async_copy(
                src_hbm.at[pl.ds(c * 8, 8)], ids.at[pl.ds(g * 8, 8)], sem0))
        for cp in cps:
            cp.wait()

        def issue(q, rd, rs, sem):
            irow = (q // 4) * 8 + (q % 4) * 2
            for t in range(2):
                pltpu.async_copy(pd_hbm.at[idd.at[irow + t]],
                                 rd.at[pl.ds(t * 80, 80)], sem)
                pltpu.async_copy(ps_hbm.at[ids.at[irow + t]],
                                 rs.at[pl.ds(t * 80, 80)], sem)

        def drain(rd, rs, sem):
            pltpu.make_async_copy(pd_hbm.at[pl.ds(0, _QR)], rd, sem).wait()
            pltpu.make_async_copy(pd_hbm.at[pl.ds(0, _QR)], rs, sem).wait()

        issue(0, *bufs[0])
        issue(1, *bufs[1])

        def body(i, _):
            for b in range(2):
                q = 2 * i + b
                rd, rs, sem = bufs[b]
                drain(rd, rs, sem)

                def addrow(r, carry):
                    for kk in range(_H // 16):
                        sl = pl.ds(kk * 16, 16)
                        rd[r, sl] = rd[r, sl] + rs[r, sl]
                    return carry

                lax.fori_loop(0, _QR, addrow, 0)
                g = q // 4
                ebase = (wid + _NW * g) * _CH + (q % 4) * _QR
                pltpu.sync_copy(rd, out_hbm.at[pl.ds(ebase, _QR)])

                @pl.when(q + 2 < nq)
                def _():
                    issue(q + 2, rd, rs, sem)
            return 0

        lax.fori_loop(0, nch_w * 2, body, 0)

    return k(Pd, Ps, dst2d, src2d)


def _sc_scatter(he, src2d, zrows):
    mesh = plsc.VectorSubcoreMesh(core_axis_name="c", subcore_axis_name="s")

    @functools.partial(
        pl.kernel, mesh=mesh,
        out_type=jax.ShapeDtypeStruct((2, _N, _H), jnp.float32),
        scratch_types=[
            pltpu.VMEM((8, 80), jnp.int32),
            pltpu.VMEM((_HCH, _H), jnp.float32),
            pltpu.VMEM_SHARED((_N, _H), jnp.float32),
        ],
    )
    def k(he_hbm, src_hbm, z_hbm, out_hbm, ids, rows, acc):
        cid = lax.axis_index("c")
        sid = lax.axis_index("s")
        wid = sid * 2 + cid
        rbase = sid * 624
        pltpu.sync_copy(z_hbm.at[pl.ds(rbase, 624)],
                        acc.at[pl.ds(rbase, 624)])

        @pl.when(sid == 0)
        def _():
            pltpu.sync_copy(z_hbm.at[pl.ds(16 * 624, _N - 16 * 624)],
                            acc.at[pl.ds(16 * 624, _N - 16 * 624)])

        plsc.subcore_barrier()

        nfull = _NCH // _NW
        nch_w = jnp.where(wid < _NCH - nfull * _NW, nfull + 1, nfull)

        def body(g, _):
            c = wid + _NW * g
            pltpu.sync_copy(src_hbm.at[pl.ds(c * 8, 8)], ids)
            for half in range(2):
                pltpu.sync_copy(
                    he_hbm.at[pl.ds(c * _CH + half * _HCH, _HCH)], rows)
                for j in range(4):
                    pltpu.sync_copy(rows.at[pl.ds(j * 80, 80)],
                                    acc.at[ids.at[half * 4 + j]], add=True)
            return 0

        lax.fori_loop(0, nch_w, body, 0)
        plsc.subcore_barrier()
        pltpu.sync_copy(acc.at[pl.ds(rbase, 624)],
                        out_hbm.at[cid, pl.ds(rbase, 624)])

        @pl.when(sid == 0)
        def _():
            pltpu.sync_copy(acc.at[pl.ds(16 * 624, _N - 16 * 624)],
                            out_hbm.at[cid, pl.ds(16 * 624, _N - 16 * 624)])

    return k(he, src2d, zrows)


def _nodes_call(body, n_out, ins):
    shapes = [jax.ShapeDtypeStruct((_N, _H), jnp.float32)] * n_out
    specs = [_row_spec((_NB, _H))] * n_out
    return pl.pallas_call(
        body,
        grid=(_N // _NB,),
        in_specs=[pl.BlockSpec((2, _NB, _H), lambda i: (0, i, 0))
                  if x.ndim == 3
                  else (_row_spec((_NB,) + x.shape[1:]) if x.shape[0] == _N
                        else _full_spec(x.shape)) for x in ins],
        out_specs=specs if n_out > 1 else specs[0],
        out_shape=shapes if n_out > 1 else shapes[0],
    )(*ins)


def _edges_call(body, ins):
    in_specs = []
    for x in ins:
        if x.shape[0] == _E:
            in_specs.append(_row_spec((_EB,) + x.shape[1:]))
        else:
            in_specs.append(_full_spec(x.shape))
    return pl.pallas_call(
        body,
        grid=(_E // _EB,),
        in_specs=in_specs,
        out_specs=_row_spec((_EB, _H)),
        out_shape=jax.ShapeDtypeStruct((_E, _H), jnp.float32),
    )(*ins)


def kernel(x, edge_index, edge_attr, p, mean_vec, std_vec, mean_edge_vec,
           std_edge_vec, ne_W1, ne_b1, ne_W2, ne_b2, ne_g, ne_b,
           ee_W1, ee_b1, ee_W2, ee_b2, ee_g, ee_b,
           pe_W1, pe_b1, pe_W2, pe_b2, pe_g, pe_b,
           pn_W1, pn_b1, pn_W2, pn_b2, pn_g, pn_b,
           dec_W1, dec_b1, dec_W2, dec_b2):
    H = _H
    f32 = jnp.float32
    src = edge_index[0]
    dst = edge_index[1]

    nW1 = ne_W1 / std_vec[:, None]
    nb1 = (ne_b1 - (mean_vec / std_vec) @ ne_W1).reshape(1, H)
    eW1 = ee_W1 / std_edge_vec[:, None]
    eb1 = (ee_b1 - (mean_edge_vec / std_edge_vec) @ ee_W1).reshape(1, H)

    wpd, wps, wpe = pe_W1[:H], pe_W1[H:2 * H], pe_W1[2 * H:]
    w1h, w1a = pn_W1[:H], pn_W1[H:]

    r2 = lambda v: v.reshape(1, -1).astype(f32)
    dW2p = jnp.zeros((H, H), f32).at[:, :dec_W2.shape[1]].set(dec_W2)
    db2p = jnp.zeros((1, H), f32).at[0, :dec_b2.shape[0]].set(dec_b2)

    h, Pd, Ps = _nodes_call(
        _enc_body, 3,
        (x, nW1, nb1, ne_W2, r2(ne_b2), r2(ne_g), r2(ne_b),
         wpd, r2(pe_b1), wps))

    pad = jnp.zeros((4096 - _E // 80, 80), jnp.int32)
    dst2d = jnp.concatenate([dst.reshape(_E // 80, 80), pad], axis=0)
    src2d = jnp.concatenate([src.reshape(_E // 80, 80), pad], axis=0)

    def gather(Pd, Ps):
        return _sc_gather(Pd, Ps, dst2d, src2d)

    zrows = jnp.zeros((_N, _H), f32)

    def scatter(he):
        return _sc_scatter(he, src2d, zrows)

    G = gather(Pd, Ps)
    he1 = _edges_call(
        _edge1_body,
        (G, edge_attr, eW1, eb1, ee_W2, r2(ee_b2), r2(ee_g), r2(ee_b),
         wpe, pe_W2, r2(pe_b2), r2(pe_g), r2(pe_b)))
    agg1 = scatter(he1)
    h1, Pd1, Ps1 = _nodes_call(
        _node_body, 3,
        (h, agg1, w1h, w1a, r2(pn_b1), pn_W2, r2(pn_b2), r2(pn_g), r2(pn_b),
         wpd, r2(pe_b1), wps))

    G1 = gather(Pd1, Ps1)
    he2 = _edges_call(
        _edge2_body,
        (G1, he1, wpe, pe_W2, r2(pe_b2), r2(pe_g), r2(pe_b)))
    agg2 = scatter(he2)

    out = _nodes_call(
        _final_body, 1,
        (h1, agg2, w1h, w1a, r2(pn_b1), pn_W2, r2(pn_b2), r2(pn_g), r2(pn_b),
         dec_W1, r2(dec_b1), dW2p, db2p))
    return out[:, :dec_W2.shape[1]]


# speedup vs baseline: 5.2990x; 1.0626x over previous
import functools

import jax
import jax.numpy as jnp
from jax import lax
from jax.experimental import pallas as pl
from jax.experimental.pallas import tpu as pltpu
from jax.experimental.pallas import tpu_sc as plsc

_N = 10000
_E = 320000
_H = 128
_NB = 1000
_EB = 4000


def _ln(t, g, b):
    mu = jnp.mean(t, axis=-1, keepdims=True)
    var = jnp.mean((t - mu) ** 2, axis=-1, keepdims=True)
    return (t - mu) * lax.rsqrt(var + 1e-5) * g + b


def _row_spec(shape):
    nd = len(shape)
    return pl.BlockSpec(shape, lambda i: (i,) + (0,) * (nd - 1))


def _full_spec(shape):
    nd = len(shape)
    return pl.BlockSpec(shape, lambda i: (0,) * nd)


def _enc_body(x_ref, w1, b1, w2, b2, g, b, wpd, bpd, wps,
              h_ref, pd_ref, ps_ref):
    t = jnp.maximum(x_ref[...] @ w1[...] + b1[...], 0.0)
    t = t @ w2[...] + b2[...]
    h = _ln(t, g[...], b[...])
    h_ref[...] = h
    pd_ref[...] = h @ wpd[...] + bpd[...]
    ps_ref[...] = h @ wps[...]


def _edge1_body(g_ref, ea_ref, ew1, eb1, ew2, eb2, eg, ebb,
                pw1e, pw2, pb2, pg, pb, out_ref):
    he = _ln(jnp.maximum(ea_ref[...] @ ew1[...] + eb1[...], 0.0)
             @ ew2[...] + eb2[...], eg[...], ebb[...])
    t = jnp.maximum(g_ref[...] + he @ pw1e[...], 0.0)
    out_ref[...] = _ln(t @ pw2[...] + pb2[...], pg[...], pb[...]) + he


def _edge2_body(g_ref, he_ref, pw1e, pw2, pb2, pg, pb, out_ref):
    he = he_ref[...]
    t = jnp.maximum(g_ref[...] + he @ pw1e[...], 0.0)
    out_ref[...] = _ln(t @ pw2[...] + pb2[...], pg[...], pb[...]) + he


def _node_body(h_ref, a_ref, w1h, w1a, b1, w2, b2, g, b, wpd, bpd, wps,
               h1_ref, pd_ref, ps_ref):
    h = h_ref[...]
    agg = a_ref[0] + a_ref[1]
    t = jnp.maximum(h @ w1h[...] + agg @ w1a[...] + b1[...], 0.0)
    t = t @ w2[...] + b2[...]
    h1 = h + _ln(t, g[...], b[...])
    h1_ref[...] = h1
    pd_ref[...] = h1 @ wpd[...] + bpd[...]
    ps_ref[...] = h1 @ wps[...]


def _final_body(h_ref, a_ref, w1h, w1a, b1, w2, b2, g, b,
                dw1, db1, dw2, db2, out_ref):
    h = h_ref[...]
    agg = a_ref[0] + a_ref[1]
    t = jnp.maximum(h @ w1h[...] + agg @ w1a[...] + b1[...], 0.0)
    t = t @ w2[...] + b2[...]
    h2 = h + _ln(t, g[...], b[...])
    o = jnp.maximum(h2 @ dw1[...] + db1[...], 0.0)
    out_ref[...] = o @ dw2[...] + db2[...]


_NW = 32
_CH = 640
_NCH = _E // _CH
_HCH = _CH // 2


_QR = 160


def _sc_gather(Pd, Ps, dst2d, src2d):
    mesh = plsc.VectorSubcoreMesh(core_axis_name="c", subcore_axis_name="s")
    nfull = _NCH // _NW
    nrem = _NCH - nfull * _NW

    @functools.partial(
        pl.kernel, mesh=mesh,
        out_type=jax.ShapeDtypeStruct((_E, _H), jnp.float32),
        scratch_types=[
            pltpu.VMEM((128, 80), jnp.int32),
            pltpu.VMEM((128, 80), jnp.int32),
            pltpu.VMEM((_QR, _H), jnp.float32),
            pltpu.VMEM((_QR, _H), jnp.float32),
            pltpu.VMEM((_QR, _H), jnp.float32),
            pltpu.VMEM((_QR, _H), jnp.float32),
            pltpu.SemaphoreType.DMA,
            pltpu.SemaphoreType.DMA,
        ],
    )
    def k(pd_hbm, ps_hbm, dst_hbm, src_hbm, out_hbm,
          idd, ids, rd0, rs0, rd1, rs1, sem0, sem1):
        wid = lax.axis_index("s") * 2 + lax.axis_index("c")
        nch_w = jnp.where(wid < nrem, nfull + 1, nfull)
        nq = nch_w * 4
        bufs = ((rd0, rs0, sem0), (rd1, rs1, sem1))

        c1 = pltpu.async_copy(
            dst_hbm.at[pl.ds(wid * 128, 128)], idd, sem0)
        c2 = pltpu.async_copy(
            src_hbm.at[pl.ds(wid * 128, 128)], ids, sem0)
        c1.wait()
        c2.wait()

        def issue(q, rd, rs, sem):
            irow = (q // 4) * 8 + (q % 4) * 2
            for t in range(2):
                pltpu.async_copy(pd_hbm.at[idd.at[irow + t]],
                                 rd.at[pl.ds(t * 80, 80)], sem)
                pltpu.async_copy(ps_hbm.at[ids.at[irow + t]],
                                 rs.at[pl.ds(t * 80, 80)], sem)

        def drain(rd, rs, sem):
            pltpu.make_async_copy(pd_hbm.at[pl.ds(0, _QR)], rd, sem).wait()
            pltpu.make_async_copy(pd_hbm.at[pl.ds(0, _QR)], rs, sem).wait()

        issue(0, *bufs[0])
        issue(1, *bufs[1])

        def body(i, _):
            for b in range(2):
                q = 2 * i + b
                rd, rs, sem = bufs[b]
                drain(rd, rs, sem)

                def addrow(r, carry):
                    for kk in range(_H // 16):
                        sl = pl.ds(kk * 16, 16)
                        rd[r, sl] = rd[r, sl] + rs[r, sl]
                    return carry

                lax.fori_loop(0, _QR, addrow, 0)
                g = q // 4
                ebase = (wid + _NW * g) * _CH + (q % 4) * _QR
                pltpu.sync_copy(rd, out_hbm.at[pl.ds(ebase, _QR)])

                @pl.when(q + 2 < nq)
                def _():
                    issue(q + 2, rd, rs, sem)
            return 0

        lax.fori_loop(0, nch_w * 2, body, 0)

    return k(Pd, Ps, dst2d, src2d)


def _sc_scatter(he, src2d, zrows):
    mesh = plsc.VectorSubcoreMesh(core_axis_name="c", subcore_axis_name="s")

    nfull = _NCH // _NW
    nrem = _NCH - nfull * _NW

    @functools.partial(
        pl.kernel, mesh=mesh,
        out_type=jax.ShapeDtypeStruct((2, _N, _H), jnp.float32),
        scratch_types=[
            pltpu.VMEM((8, 80), jnp.int32),
            pltpu.VMEM((_QR, _H), jnp.float32),
            pltpu.VMEM((_QR, _H), jnp.float32),
            pltpu.VMEM_SHARED((_N, _H), jnp.float32),
            pltpu.SemaphoreType.DMA,
            pltpu.SemaphoreType.DMA,
            pltpu.SemaphoreType.DMA,
            pltpu.SemaphoreType.DMA,
        ],
    )
    def k(he_hbm, src_hbm, z_hbm, out_hbm, ids, rows0, rows1, acc,
          rsem0, rsem1, asem0, asem1):
        cid = lax.axis_index("c")
        sid = lax.axis_index("s")
        wid = sid * 2 + cid
        rows = (rows0, rows1)
        rsem = (rsem0, rsem1)
        asem = (asem0, asem1)
        rbase = sid * 624
        pltpu.sync_copy(z_hbm.at[pl.ds(rbase, 624)],
                        acc.at[pl.ds(rbase, 624)])

        @pl.when(sid == 0)
        def _():
            pltpu.sync_copy(z_hbm.at[pl.ds(16 * 624, _N - 16 * 624)],
                            acc.at[pl.ds(16 * 624, _N - 16 * 624)])

        plsc.subcore_barrier()
        nch_w = jnp.where(wid < nrem, nfull + 1, nfull)

        def read(g, r, b):
            ebase = (wid + _NW * g) * _CH + r * _QR
            pltpu.async_copy(he_hbm.at[pl.ds(ebase, _QR)], rows[b], rsem[b])

        def drain_read(b):
            pltpu.make_async_copy(
                he_hbm.at[pl.ds(0, _QR)], rows[b], rsem[b]).wait()

        def drain_adds(b):
            pltpu.make_async_copy(
                rows[b], acc.at[pl.ds(0, _QR)], asem[b]).wait()

        read(0, 0, 0)
        read(0, 1, 1)

        def body(g, _):
            @pl.when(g > 0)
            def _():
                drain_adds(1)
                read(g, 1, 1)

            pltpu.sync_copy(src_hbm.at[pl.ds((wid + _NW * g) * 8, 8)], ids)
            for r in range(4):
                b = r % 2
                drain_read(b)
                for t in range(2):
                    pltpu.async_copy(rows[b].at[pl.ds(t * 80, 80)],
                                     acc.at[ids.at[r * 2 + t]],
                                     asem[b], add=True)
                if r >= 1:
                    ob = 1 - b
                    drain_adds(ob)
                    if r < 3:
                        read(g, r + 1, ob)
                    else:
                        @pl.when(g + 1 < nch_w)
                        def _():
                            read(g + 1, 0, ob)
            return 0

        lax.fori_loop(0, nch_w, body, 0)
        drain_adds(1)
        plsc.subcore_barrier()
        pltpu.sync_copy(acc.at[pl.ds(rbase, 624)],
                        out_hbm.at[cid, pl.ds(rbase, 624)])

        @pl.when(sid == 0)
        def _():
            pltpu.sync_copy(acc.at[pl.ds(16 * 624, _N - 16 * 624)],
                            out_hbm.at[cid, pl.ds(16 * 624, _N - 16 * 624)])

    return k(he, src2d, zrows)


def _nodes_call(body, n_out, ins):
    shapes = [jax.ShapeDtypeStruct((_N, _H), jnp.float32)] * n_out
    specs = [_row_spec((_NB, _H))] * n_out
    return pl.pallas_call(
        body,
        grid=(_N // _NB,),
        in_specs=[pl.BlockSpec((2, _NB, _H), lambda i: (0, i, 0))
                  if x.ndim == 3
                  else (_row_spec((_NB,) + x.shape[1:]) if x.shape[0] == _N
                        else _full_spec(x.shape)) for x in ins],
        out_specs=specs if n_out > 1 else specs[0],
        out_shape=shapes if n_out > 1 else shapes[0],
    )(*ins)


def _edges_call(body, ins):
    in_specs = []
    for x in ins:
        if x.shape[0] == _E:
            in_specs.append(_row_spec((_EB,) + x.shape[1:]))
        else:
            in_specs.append(_full_spec(x.shape))
    return pl.pallas_call(
        body,
        grid=(_E // _EB,),
        in_specs=in_specs,
        out_specs=_row_spec((_EB, _H)),
        out_shape=jax.ShapeDtypeStruct((_E, _H), jnp.float32),
    )(*ins)


def kernel(x, edge_index, edge_attr, p, mean_vec, std_vec, mean_edge_vec,
           std_edge_vec, ne_W1, ne_b1, ne_W2, ne_b2, ne_g, ne_b,
           ee_W1, ee_b1, ee_W2, ee_b2, ee_g, ee_b,
           pe_W1, pe_b1, pe_W2, pe_b2, pe_g, pe_b,
           pn_W1, pn_b1, pn_W2, pn_b2, pn_g, pn_b,
           dec_W1, dec_b1, dec_W2, dec_b2):
    H = _H
    f32 = jnp.float32
    src = edge_index[0]
    dst = edge_index[1]

    nW1 = ne_W1 / std_vec[:, None]
    nb1 = (ne_b1 - (mean_vec / std_vec) @ ne_W1).reshape(1, H)
    eW1 = ee_W1 / std_edge_vec[:, None]
    eb1 = (ee_b1 - (mean_edge_vec / std_edge_vec) @ ee_W1).reshape(1, H)

    wpd, wps, wpe = pe_W1[:H], pe_W1[H:2 * H], pe_W1[2 * H:]
    w1h, w1a = pn_W1[:H], pn_W1[H:]

    r2 = lambda v: v.reshape(1, -1).astype(f32)
    dW2p = jnp.zeros((H, H), f32).at[:, :dec_W2.shape[1]].set(dec_W2)
    db2p = jnp.zeros((1, H), f32).at[0, :dec_b2.shape[0]].set(dec_b2)

    h, Pd, Ps = _nodes_call(
        _enc_body, 3,
        (x, nW1, nb1, ne_W2, r2(ne_b2), r2(ne_g), r2(ne_b),
         wpd, r2(pe_b1), wps))

    pad = jnp.zeros((4096 - _E // 80, 80), jnp.int32)
    dst2d = jnp.concatenate([dst.reshape(_E // 80, 80), pad], axis=0)
    src2d = jnp.concatenate([src.reshape(_E // 80, 80), pad], axis=0)
    perm = (jnp.arange(_NW)[:, None] + _NW * jnp.arange(16)[None, :]
            ).reshape(-1)
    dstp = dst2d.reshape(512, 8, 80)[perm].reshape(4096, 80)
    srcp = src2d.reshape(512, 8, 80)[perm].reshape(4096, 80)

    def gather(Pd, Ps):
        return _sc_gather(Pd, Ps, dstp, srcp)

    zrows = jnp.zeros((_N, _H), f32)

    def scatter(he):
        return _sc_scatter(he, src2d, zrows)

    G = gather(Pd, Ps)
    he1 = _edges_call(
        _edge1_body,
        (G, edge_attr, eW1, eb1, ee_W2, r2(ee_b2), r2(ee_g), r2(ee_b),
         wpe, pe_W2, r2(pe_b2), r2(pe_g), r2(pe_b)))
    agg1 = scatter(he1)
    h1, Pd1, Ps1 = _nodes_call(
        _node_body, 3,
        (h, agg1, w1h, w1a, r2(pn_b1), pn_W2, r2(pn_b2), r2(pn_g), r2(pn_b),
         wpd, r2(pe_b1), wps))

    G1 = gather(Pd1, Ps1)
    he2 = _edges_call(
        _edge2_body,
        (G1, he1, wpe, pe_W2, r2(pe_b2), r2(pe_g), r2(pe_b)))
    agg2 = scatter(he2)

    out = _nodes_call(
        _final_body, 1,
        (h1, agg2, w1h, w1a, r2(pn_b1), pn_W2, r2(pn_b2), r2(pn_g), r2(pn_b),
         dec_W1, r2(dec_b1), dW2p, db2p))
    return out[:, :dec_W2.shape[1]]
